# Initial kernel scaffold; baseline (speedup 1.0000x reference)
import functools

import jax
import jax.numpy as jnp
from jax import lax
from jax.experimental import pallas as pl
from jax.experimental.pallas import tpu as pltpu
from jax.experimental.pallas import tpu_sc as plsc

N = 10000
NE = 320000
DIN, DH, DOUT = 128, 128, 64
NC, NS = 2, 16
NW = NC * NS
CH = 128
NCH = 80
NEP = NW * NCH * CH
NROWS = 10240
STRIPE = NROWS // NS
DEGW = 16
RB = 1000


def _sc_mesh():
    return plsc.VectorSubcoreMesh(core_axis_name="c", subcore_axis_name="s")


def _make_degree_kernel():
    @functools.partial(
        pl.kernel,
        out_type=jax.ShapeDtypeStruct((NC, NROWS, DEGW), jnp.float32),
        mesh=_sc_mesh(),
        scratch_types=[
            pltpu.VMEM((CH,), jnp.int32),
            pltpu.VMEM((CH, DEGW), jnp.float32),
            pltpu.VMEM_SHARED((NROWS, DEGW), jnp.float32),
        ],
    )
    def deg_kernel(dstp, ones, zeros, out, idx_d, onesv, acc):
        c = lax.axis_index("c")
        s = lax.axis_index("s")
        w = c * NS + s
        pltpu.sync_copy(zeros, acc.at[pl.ds(s * STRIPE, STRIPE)])
        pltpu.sync_copy(ones, onesv)
        plsc.subcore_barrier()

        def body(j, carry):
            pltpu.sync_copy(dstp.at[w, j], idx_d)
            pltpu.sync_copy(onesv, acc.at[idx_d], add=True)
            return carry

        lax.fori_loop(0, NCH, body, 0)
        plsc.subcore_barrier()
        pltpu.sync_copy(acc.at[pl.ds(s * STRIPE, STRIPE)],
                        out.at[c, pl.ds(s * STRIPE, STRIPE)])

    return deg_kernel


def _make_gather_scatter_kernel(D):
    @functools.partial(
        pl.kernel,
        out_type=jax.ShapeDtypeStruct((NC, NROWS, D), jnp.float32),
        mesh=_sc_mesh(),
        scratch_types=[
            pltpu.VMEM((CH,), jnp.int32),
            pltpu.VMEM((CH,), jnp.int32),
            pltpu.VMEM((CH, D), jnp.float32),
            pltpu.VMEM_SHARED((NROWS, D), jnp.float32),
            pltpu.SemaphoreType.DMA,
        ],
    )
    def gs_kernel(table, srcp, dstp, zeros, out, idx_s, idx_d, rows, acc, sem):
        c = lax.axis_index("c")
        s = lax.axis_index("s")
        w = c * NS + s
        pltpu.sync_copy(zeros, acc.at[pl.ds(s * STRIPE, STRIPE)])
        plsc.subcore_barrier()

        def body(j, carry):
            pltpu.sync_copy(srcp.at[w, j], idx_s)
            pltpu.sync_copy(dstp.at[w, j], idx_d)
            pltpu.async_copy(table.at[idx_s], rows, sem).wait()
            pltpu.sync_copy(rows, acc.at[idx_d], add=True)
            return carry

        lax.fori_loop(0, NCH, body, 0)
        plsc.subcore_barrier()
        pltpu.sync_copy(acc.at[pl.ds(s * STRIPE, STRIPE)],
                        out.at[c, pl.ds(s * STRIPE, STRIPE)])

    return gs_kernel


def _dinv_block(deg_ref):
    deg = deg_ref[0, :, 0:1] + deg_ref[1, :, 0:1]
    return lax.rsqrt(deg)


def _tc1_body(deg_ref, x_ref, w_ref, h_ref, hp_ref):
    dinv = _dinv_block(deg_ref)
    h = jnp.dot(x_ref[...], w_ref[...], preferred_element_type=jnp.float32)
    h_ref[...] = h
    hp_ref[...] = h * dinv


def _tc2_body(deg_ref, s_ref, h1_ref, b_ref, w_ref, h2_ref, hp2_ref):
    dinv = _dinv_block(deg_ref)
    sagg = s_ref[0] + s_ref[1]
    a = dinv * sagg + (dinv * dinv) * h1_ref[...] + b_ref[...]
    a = jnp.maximum(a, 0.0)
    h2 = jnp.dot(a, w_ref[...], preferred_element_type=jnp.float32)
    h2_ref[...] = h2
    hp2_ref[...] = h2 * dinv


def _tc3_body(deg_ref, s_ref, h2_ref, b_ref, o_ref):
    dinv = _dinv_block(deg_ref)
    sagg = s_ref[0] + s_ref[1]
    z = dinv * sagg + (dinv * dinv) * h2_ref[...] + b_ref[...]
    m = jnp.max(z, axis=1, keepdims=True)
    ez = jnp.exp(z - m)
    lse = jnp.log(jnp.sum(ez, axis=1, keepdims=True)) + m
    o_ref[...] = z - lse


def _deg_spec():
    return pl.BlockSpec((NC, RB, DEGW), lambda i: (0, i, 0))


def _acc_spec(D):
    return pl.BlockSpec((NC, RB, D), lambda i: (0, i, 0))


def _full_spec(r, c):
    return pl.BlockSpec((r, c), lambda i: (0, 0))


def _row_spec(D):
    return pl.BlockSpec((RB, D), lambda i: (i, 0))


def kernel(x, edge_index, W1, b1, W2, b2):
    src = edge_index[0].astype(jnp.int32)
    dst = edge_index[1].astype(jnp.int32)
    npad = NEP - NE
    pad = jnp.arange(npad, dtype=jnp.int32)
    src_p = jnp.concatenate([src, pad % N]).reshape(NW, NCH, CH)
    dst_p = jnp.concatenate([dst, N + pad % (NROWS - N)]).reshape(NW, NCH, CH)

    ones = jnp.ones((CH, DEGW), jnp.float32)
    zeros_deg = jnp.zeros((STRIPE, DEGW), jnp.float32)
    zeros_h = jnp.zeros((STRIPE, DH), jnp.float32)
    zeros_o = jnp.zeros((STRIPE, DOUT), jnp.float32)

    degp = _make_degree_kernel()(dst_p, ones, zeros_deg)

    grid = (N // RB,)

    h1, hp1 = pl.pallas_call(
        _tc1_body,
        grid=grid,
        in_specs=[_deg_spec(), _row_spec(DIN), _full_spec(DIN, DH)],
        out_specs=[_row_spec(DH), _row_spec(DH)],
        out_shape=[jax.ShapeDtypeStruct((N, DH), jnp.float32)] * 2,
    )(degp, x, W1)

    s1p = _make_gather_scatter_kernel(DH)(hp1, src_p, dst_p, zeros_h)

    h2, hp2 = pl.pallas_call(
        _tc2_body,
        grid=grid,
        in_specs=[_deg_spec(), _acc_spec(DH), _row_spec(DH),
                  _full_spec(1, DH), _full_spec(DH, DOUT)],
        out_specs=[_row_spec(DOUT), _row_spec(DOUT)],
        out_shape=[jax.ShapeDtypeStruct((N, DOUT), jnp.float32)] * 2,
    )(degp, s1p, h1, b1.reshape(1, DH), W2)

    s2p = _make_gather_scatter_kernel(DOUT)(hp2, src_p, dst_p, zeros_o)

    out = pl.pallas_call(
        _tc3_body,
        grid=grid,
        in_specs=[_deg_spec(), _acc_spec(DOUT), _row_spec(DOUT),
                  _full_spec(1, DOUT)],
        out_specs=_row_spec(DOUT),
        out_shape=jax.ShapeDtypeStruct((N, DOUT), jnp.float32),
    )(degp, s2p, h2, b2.reshape(1, DOUT))

    return out


# speedup vs baseline: 16.5805x; 16.5805x over previous
import functools

import jax
import jax.numpy as jnp
from jax import lax
from jax.experimental import pallas as pl
from jax.experimental.pallas import tpu as pltpu
from jax.experimental.pallas import tpu_sc as plsc

N = 10000
NE = 320000
DIN, DH, DOUT = 128, 128, 64
NC, NS = 2, 16
NW = NC * NS
CH = 128
NCH = 80
NEP = NW * NCH * CH
NROWS = 10240
STRIPE = NROWS // NS
DEGW = 16
RB = 1000


def _sc_mesh():
    return plsc.VectorSubcoreMesh(core_axis_name="c", subcore_axis_name="s")


def _make_degree_kernel():
    @functools.partial(
        pl.kernel,
        out_type=jax.ShapeDtypeStruct((NC, NROWS, DEGW), jnp.float32),
        mesh=_sc_mesh(),
        scratch_types=[
            pltpu.VMEM((CH,), jnp.int32),
            pltpu.VMEM((CH, DEGW), jnp.float32),
            pltpu.VMEM_SHARED((NROWS, DEGW), jnp.float32),
        ],
    )
    def deg_kernel(dstp, ones, zeros, out, idx_d, onesv, acc):
        c = lax.axis_index("c")
        s = lax.axis_index("s")
        w = c * NS + s
        pltpu.sync_copy(zeros, acc.at[pl.ds(s * STRIPE, STRIPE)])
        pltpu.sync_copy(ones, onesv)
        plsc.subcore_barrier()

        def body(j, carry):
            pltpu.sync_copy(dstp.at[w, j], idx_d)
            pltpu.sync_copy(onesv, acc.at[idx_d], add=True)
            return carry

        lax.fori_loop(0, NCH, body, 0)
        plsc.subcore_barrier()
        pltpu.sync_copy(acc.at[pl.ds(s * STRIPE, STRIPE)],
                        out.at[c, pl.ds(s * STRIPE, STRIPE)])

    return deg_kernel


def _make_gather_scatter_kernel(D):
    @functools.partial(
        pl.kernel,
        out_type=jax.ShapeDtypeStruct((NC, NROWS, D), jnp.float32),
        mesh=_sc_mesh(),
        compiler_params=pltpu.CompilerParams(use_tc_tiling_on_sc=False),
        scratch_types=[
            pltpu.VMEM((CH,), jnp.int32),
            pltpu.VMEM((CH,), jnp.int32),
            pltpu.VMEM((CH, D), jnp.float32),
            pltpu.VMEM_SHARED((NROWS, D), jnp.float32),
            pltpu.SemaphoreType.DMA,
        ],
    )
    def gs_kernel(table, srcp, dstp, zeros, out, idx_s, idx_d, rows, acc, sem):
        c = lax.axis_index("c")
        s = lax.axis_index("s")
        w = c * NS + s
        pltpu.sync_copy(zeros, acc.at[pl.ds(s * STRIPE, STRIPE)])
        plsc.subcore_barrier()

        def body(j, carry):
            pltpu.sync_copy(srcp.at[w, j], idx_s)
            pltpu.sync_copy(dstp.at[w, j], idx_d)
            pltpu.async_copy(table.at[idx_s], rows, sem).wait()
            pltpu.sync_copy(rows, acc.at[idx_d], add=True)
            return carry

        lax.fori_loop(0, NCH, body, 0)
        plsc.subcore_barrier()
        pltpu.sync_copy(acc.at[pl.ds(s * STRIPE, STRIPE)],
                        out.at[c, pl.ds(s * STRIPE, STRIPE)])

    return gs_kernel


def _dinv_block(deg_ref):
    deg = deg_ref[0, :, 0:1] + deg_ref[1, :, 0:1]
    return lax.rsqrt(deg)


def _tc1_body(deg_ref, x_ref, w_ref, h_ref, hp_ref):
    dinv = _dinv_block(deg_ref)
    h = jnp.dot(x_ref[...], w_ref[...], preferred_element_type=jnp.float32)
    h_ref[...] = h
    hp_ref[...] = h * dinv


def _tc2_body(deg_ref, s_ref, h1_ref, b_ref, w_ref, h2_ref, hp2_ref):
    dinv = _dinv_block(deg_ref)
    sagg = s_ref[0] + s_ref[1]
    a = dinv * sagg + (dinv * dinv) * h1_ref[...] + b_ref[...]
    a = jnp.maximum(a, 0.0)
    h2 = jnp.dot(a, w_ref[...], preferred_element_type=jnp.float32)
    h2_ref[...] = h2
    hp2_ref[...] = h2 * dinv


def _tc3_body(deg_ref, s_ref, h2_ref, b_ref, o_ref):
    dinv = _dinv_block(deg_ref)
    sagg = s_ref[0] + s_ref[1]
    z = dinv * sagg + (dinv * dinv) * h2_ref[...] + b_ref[...]
    m = jnp.max(z, axis=1, keepdims=True)
    ez = jnp.exp(z - m)
    lse = jnp.log(jnp.sum(ez, axis=1, keepdims=True)) + m
    o_ref[...] = z - lse


def _deg_spec():
    return pl.BlockSpec((NC, RB, DEGW), lambda i: (0, i, 0))


def _acc_spec(D):
    return pl.BlockSpec((NC, RB, D), lambda i: (0, i, 0))


def _full_spec(r, c):
    return pl.BlockSpec((r, c), lambda i: (0, 0))


def _row_spec(D):
    return pl.BlockSpec((RB, D), lambda i: (i, 0))


def kernel(x, edge_index, W1, b1, W2, b2):
    src = edge_index[0].astype(jnp.int32)
    dst = edge_index[1].astype(jnp.int32)
    npad = NEP - NE
    pad = jnp.arange(npad, dtype=jnp.int32)
    src_p = jnp.concatenate([src, pad % N]).reshape(NW, NCH, CH)
    dst_p = jnp.concatenate([dst, N + pad % (NROWS - N)]).reshape(NW, NCH, CH)

    ones = jnp.ones((CH, DEGW), jnp.float32)
    zeros_deg = jnp.zeros((STRIPE, DEGW), jnp.float32)
    zeros_h = jnp.zeros((STRIPE, DH), jnp.float32)
    zeros_o = jnp.zeros((STRIPE, DOUT), jnp.float32)

    degp = _make_degree_kernel()(dst_p, ones, zeros_deg)

    grid = (N // RB,)

    h1, hp1 = pl.pallas_call(
        _tc1_body,
        grid=grid,
        in_specs=[_deg_spec(), _row_spec(DIN), _full_spec(DIN, DH)],
        out_specs=[_row_spec(DH), _row_spec(DH)],
        out_shape=[jax.ShapeDtypeStruct((N, DH), jnp.float32)] * 2,
    )(degp, x, W1)

    s1p = _make_gather_scatter_kernel(DH)(hp1, src_p, dst_p, zeros_h)

    h2, hp2 = pl.pallas_call(
        _tc2_body,
        grid=grid,
        in_specs=[_deg_spec(), _acc_spec(DH), _row_spec(DH),
                  _full_spec(1, DH), _full_spec(DH, DOUT)],
        out_specs=[_row_spec(DOUT), _row_spec(DOUT)],
        out_shape=[jax.ShapeDtypeStruct((N, DOUT), jnp.float32)] * 2,
    )(degp, s1p, h1, b1.reshape(1, DH), W2)

    s2p = _make_gather_scatter_kernel(DOUT)(hp2, src_p, dst_p, zeros_o)

    out = pl.pallas_call(
        _tc3_body,
        grid=grid,
        in_specs=[_deg_spec(), _acc_spec(DOUT), _row_spec(DOUT),
                  _full_spec(1, DOUT)],
        out_specs=_row_spec(DOUT),
        out_shape=jax.ShapeDtypeStruct((N, DOUT), jnp.float32),
    )(degp, s2p, h2, b2.reshape(1, DOUT))

    return out


# speedup vs baseline: 24.7232x; 1.4911x over previous
import functools

import jax
import jax.numpy as jnp
from jax import lax
from jax.experimental import pallas as pl
from jax.experimental.pallas import tpu as pltpu
from jax.experimental.pallas import tpu_sc as plsc

N = 10000
NE = 320000
DIN, DH, DOUT = 128, 128, 64
DHH = DH // 2
NC, NS = 2, 16
NW = NC * NS
CH = 128
NCH = 80
NCH2 = NCH * NC
NEP = NW * NCH * CH
NROWS = 10112
STRIPE = NROWS // NS
DEGW = 8
RB = 1000


def _sc_mesh():
    return plsc.VectorSubcoreMesh(core_axis_name="c", subcore_axis_name="s")


def _make_degree_kernel():
    @functools.partial(
        pl.kernel,
        out_type=jax.ShapeDtypeStruct((NC, NROWS, DEGW), jnp.float32),
        mesh=_sc_mesh(),
        scratch_types=[
            pltpu.VMEM((NCH, CH), jnp.int32),
            pltpu.VMEM((CH, DEGW), jnp.float32),
            pltpu.VMEM_SHARED((NROWS, DEGW), jnp.float32),
        ],
    )
    def deg_kernel(dstp, ones, zeros, out, idx_d, onesv, acc):
        c = lax.axis_index("c")
        s = lax.axis_index("s")
        w = c * NS + s
        pltpu.sync_copy(dstp.at[w], idx_d)
        pltpu.sync_copy(ones, onesv)
        pltpu.sync_copy(zeros, acc.at[pl.ds(s * STRIPE, STRIPE)])
        plsc.subcore_barrier()

        def body(j, carry):
            pltpu.sync_copy(onesv, acc.at[idx_d.at[j]], add=True)
            return carry

        lax.fori_loop(0, NCH, body, 0)
        plsc.subcore_barrier()
        pltpu.sync_copy(acc.at[pl.ds(s * STRIPE, STRIPE)],
                        out.at[c, pl.ds(s * STRIPE, STRIPE)])

    return deg_kernel


def _pipelined_chunk_loop(table, idx_s, idx_d, rows, acc, sems, n_chunks):
    sg = [sems[0], sems[1]]
    ss = [sems[2], sems[3]]
    pltpu.async_copy(table.at[idx_s.at[0]], rows.at[0], sg[0])

    def body(j, carry):
        for p in (0, 1):
            @pl.when(j % 2 == p)
            def _():
                q = 1 - p
                pltpu.make_async_copy(
                    table.at[idx_s.at[0]], rows.at[p], sg[p]).wait()

                @pl.when(j + 1 < n_chunks)
                def _():
                    @pl.when(j >= 1)
                    def _():
                        pltpu.make_async_copy(
                            rows.at[q], acc.at[idx_d.at[0]], ss[q]).wait()
                    pltpu.async_copy(table.at[idx_s.at[j + 1]],
                                     rows.at[q], sg[q])
                pltpu.async_copy(rows.at[p], acc.at[idx_d.at[j]],
                                 ss[p], add=True)
        return carry

    lax.fori_loop(0, n_chunks, body, 0)
    for sem in ss:
        pltpu.make_async_copy(rows.at[0], acc.at[idx_d.at[0]], sem).wait()


def _make_colsplit_kernel():
    @functools.partial(
        pl.kernel,
        out_type=jax.ShapeDtypeStruct((NC, NROWS, DHH), jnp.float32),
        mesh=_sc_mesh(),
        compiler_params=pltpu.CompilerParams(use_tc_tiling_on_sc=False),
        scratch_types=[
            pltpu.VMEM((NCH2, CH), jnp.int32),
            pltpu.VMEM((NCH2, CH), jnp.int32),
            pltpu.VMEM((2, CH, DHH), jnp.float32),
            pltpu.VMEM_SHARED((NROWS, DHH), jnp.float32),
        ] + [pltpu.SemaphoreType.DMA] * 4,
    )
    def gs_kernel(table2, srcp, dstp, zeros, out, idx_s, idx_d, rows, acc,
                  *sems):
        c = lax.axis_index("c")
        s = lax.axis_index("s")
        pltpu.sync_copy(srcp.at[s], idx_s)
        pltpu.sync_copy(dstp.at[s], idx_d)
        pltpu.sync_copy(zeros, acc.at[pl.ds(s * STRIPE, STRIPE)])
        plsc.subcore_barrier()
        _pipelined_chunk_loop(table2.at[c], idx_s, idx_d, rows, acc, sems,
                              NCH2)
        plsc.subcore_barrier()
        pltpu.sync_copy(acc.at[pl.ds(s * STRIPE, STRIPE)],
                        out.at[c, pl.ds(s * STRIPE, STRIPE)])

    return gs_kernel


def _make_edgesplit_kernel(D):
    @functools.partial(
        pl.kernel,
        out_type=jax.ShapeDtypeStruct((NC, NROWS, D), jnp.float32),
        mesh=_sc_mesh(),
        compiler_params=pltpu.CompilerParams(use_tc_tiling_on_sc=False),
        scratch_types=[
            pltpu.VMEM((NCH, CH), jnp.int32),
            pltpu.VMEM((NCH, CH), jnp.int32),
            pltpu.VMEM((2, CH, D), jnp.float32),
            pltpu.VMEM_SHARED((NROWS, D), jnp.float32),
        ] + [pltpu.SemaphoreType.DMA] * 4,
    )
    def gs_kernel(table, srcp, dstp, zeros, out, idx_s, idx_d, rows, acc,
                  *sems):
        c = lax.axis_index("c")
        s = lax.axis_index("s")
        w = c * NS + s
        pltpu.sync_copy(srcp.at[w], idx_s)
        pltpu.sync_copy(dstp.at[w], idx_d)
        pltpu.sync_copy(zeros, acc.at[pl.ds(s * STRIPE, STRIPE)])
        plsc.subcore_barrier()
        _pipelined_chunk_loop(table, idx_s, idx_d, rows, acc, sems, NCH)
        plsc.subcore_barrier()
        pltpu.sync_copy(acc.at[pl.ds(s * STRIPE, STRIPE)],
                        out.at[c, pl.ds(s * STRIPE, STRIPE)])

    return gs_kernel


def _dinv_block(deg_ref):
    deg = deg_ref[0, :, 0:1] + deg_ref[1, :, 0:1]
    return lax.rsqrt(deg)


def _tc1_body(deg_ref, x_ref, w_ref, h_ref, hp_ref):
    dinv = _dinv_block(deg_ref)
    h = jnp.dot(x_ref[...], w_ref[...], preferred_element_type=jnp.float32)
    h_ref[...] = h
    hp = h * dinv
    hp_ref[0] = hp[:, :DHH]
    hp_ref[1] = hp[:, DHH:]


def _tc2_body(deg_ref, s_ref, h1_ref, b_ref, w_ref, h2_ref, hp2_ref):
    dinv = _dinv_block(deg_ref)
    sagg = jnp.concatenate([s_ref[0], s_ref[1]], axis=1)
    a = dinv * sagg + (dinv * dinv) * h1_ref[...] + b_ref[...]
    a = jnp.maximum(a, 0.0)
    h2 = jnp.dot(a, w_ref[...], preferred_element_type=jnp.float32)
    h2_ref[...] = h2
    hp2_ref[...] = h2 * dinv


def _tc3_body(deg_ref, s_ref, h2_ref, b_ref, o_ref):
    dinv = _dinv_block(deg_ref)
    sagg = s_ref[0] + s_ref[1]
    z = dinv * sagg + (dinv * dinv) * h2_ref[...] + b_ref[...]
    m = jnp.max(z, axis=1, keepdims=True)
    ez = jnp.exp(z - m)
    lse = jnp.log(jnp.sum(ez, axis=1, keepdims=True)) + m
    o_ref[...] = z - lse


def _deg_spec():
    return pl.BlockSpec((NC, RB, DEGW), lambda i: (0, i, 0))


def _acc_spec(D):
    return pl.BlockSpec((NC, RB, D), lambda i: (0, i, 0))


def _full_spec(r, c):
    return pl.BlockSpec((r, c), lambda i: (0, 0))


def _row_spec(D):
    return pl.BlockSpec((RB, D), lambda i: (i, 0))


def kernel(x, edge_index, W1, b1, W2, b2):
    src = edge_index[0].astype(jnp.int32)
    dst = edge_index[1].astype(jnp.int32)
    npad = NEP - NE
    pad = jnp.arange(npad, dtype=jnp.int32)
    src_pad = jnp.concatenate([src, pad % N])
    dst_pad = jnp.concatenate([dst, N + pad % (NROWS - N)])
    src_p = src_pad.reshape(NW, NCH, CH)
    dst_p = dst_pad.reshape(NW, NCH, CH)
    src_p2 = src_pad.reshape(NS, NCH2, CH)
    dst_p2 = dst_pad.reshape(NS, NCH2, CH)

    ones = jnp.ones((CH, DEGW), jnp.float32)
    zeros_deg = jnp.zeros((STRIPE, DEGW), jnp.float32)
    zeros_h = jnp.zeros((STRIPE, DHH), jnp.float32)
    zeros_o = jnp.zeros((STRIPE, DOUT), jnp.float32)

    degp = _make_degree_kernel()(dst_p, ones, zeros_deg)

    grid = (N // RB,)

    h1, hp1 = pl.pallas_call(
        _tc1_body,
        grid=grid,
        in_specs=[_deg_spec(), _row_spec(DIN), _full_spec(DIN, DH)],
        out_specs=[_row_spec(DH),
                   pl.BlockSpec((NC, RB, DHH), lambda i: (0, i, 0))],
        out_shape=[jax.ShapeDtypeStruct((N, DH), jnp.float32),
                   jax.ShapeDtypeStruct((NC, N, DHH), jnp.float32)],
    )(degp, x, W1)

    s1p = _make_colsplit_kernel()(hp1, src_p2, dst_p2, zeros_h)

    h2, hp2 = pl.pallas_call(
        _tc2_body,
        grid=grid,
        in_specs=[_deg_spec(), _acc_spec(DHH), _row_spec(DH),
                  _full_spec(1, DH), _full_spec(DH, DOUT)],
        out_specs=[_row_spec(DOUT), _row_spec(DOUT)],
        out_shape=[jax.ShapeDtypeStruct((N, DOUT), jnp.float32)] * 2,
    )(degp, s1p, h1, b1.reshape(1, DH), W2)

    s2p = _make_edgesplit_kernel(DOUT)(hp2, src_p, dst_p, zeros_o)

    out = pl.pallas_call(
        _tc3_body,
        grid=grid,
        in_specs=[_deg_spec(), _acc_spec(DOUT), _row_spec(DOUT),
                  _full_spec(1, DOUT)],
        out_specs=_row_spec(DOUT),
        out_shape=jax.ShapeDtypeStruct((N, DOUT), jnp.float32),
    )(degp, s2p, h2, b2.reshape(1, DOUT))

    return out


# speedup vs baseline: 30.8807x; 1.2491x over previous
import functools

import jax
import jax.numpy as jnp
from jax import lax
from jax.experimental import pallas as pl
from jax.experimental.pallas import tpu as pltpu
from jax.experimental.pallas import tpu_sc as plsc

N = 10000
NE = 320000
DIN, DH, DOUT = 128, 128, 64
DHH = DH // 2
NC, NS = 2, 16
NW = NC * NS
CH = 128
NCH = 80
NCH2 = NCH * NC
NEP = NW * NCH * CH
NROWS = 10112
STRIPE = NROWS // NS
DEGW = 8
RB = 1000


def _sc_mesh():
    return plsc.VectorSubcoreMesh(core_axis_name="c", subcore_axis_name="s")


def _make_degree_kernel():
    @functools.partial(
        pl.kernel,
        out_type=jax.ShapeDtypeStruct((NC, NROWS, DEGW), jnp.float32),
        mesh=_sc_mesh(),
        scratch_types=[
            pltpu.VMEM((NCH, CH), jnp.int32),
            pltpu.VMEM((CH, DEGW), jnp.float32),
            pltpu.VMEM_SHARED((NROWS, DEGW), jnp.float32),
        ] + [pltpu.SemaphoreType.DMA] * 2,
    )
    def deg_kernel(dstp, ones, zeros, out, idx_d, onesv, acc, *sems):
        c = lax.axis_index("c")
        s = lax.axis_index("s")
        w = c * NS + s
        pltpu.sync_copy(dstp.at[w], idx_d)
        pltpu.sync_copy(ones, onesv)
        pltpu.sync_copy(zeros, acc.at[pl.ds(s * STRIPE, STRIPE)])
        plsc.subcore_barrier()

        def body(j, carry):
            for p in (0, 1):
                @pl.when(j % 2 == p)
                def _():
                    @pl.when(j >= 2)
                    def _():
                        pltpu.make_async_copy(
                            onesv, acc.at[idx_d.at[0]], sems[p]).wait()
                    pltpu.async_copy(onesv, acc.at[idx_d.at[j]], sems[p],
                                     add=True)
            return carry

        lax.fori_loop(0, NCH, body, 0)
        for sem in sems:
            pltpu.make_async_copy(onesv, acc.at[idx_d.at[0]], sem).wait()
        plsc.subcore_barrier()
        pltpu.sync_copy(acc.at[pl.ds(s * STRIPE, STRIPE)],
                        out.at[c, pl.ds(s * STRIPE, STRIPE)])

    return deg_kernel


def _pipelined_chunk_loop(table, idx_s, idx_d, rows, acc, sems, n_chunks):
    sg = [sems[0], sems[1]]
    ss = [sems[2], sems[3]]
    pltpu.async_copy(table.at[idx_s.at[0]], rows.at[0], sg[0])

    @pl.when(n_chunks > 1)
    def _():
        pltpu.async_copy(table.at[idx_s.at[1]], rows.at[1], sg[1])

    def body(j, carry):
        for p in (0, 1, 2, 3):
            @pl.when(j % 4 == p)
            def _():
                h = p % 2
                pltpu.make_async_copy(
                    table.at[idx_s.at[0]], rows.at[p], sg[h]).wait()

                @pl.when(j + 2 < n_chunks)
                def _():
                    @pl.when(j >= 2)
                    def _():
                        pltpu.make_async_copy(
                            rows.at[(p + 2) % 4], acc.at[idx_d.at[0]],
                            ss[h]).wait()
                    pltpu.async_copy(table.at[idx_s.at[j + 2]],
                                     rows.at[(p + 2) % 4], sg[h])
                pltpu.async_copy(rows.at[p], acc.at[idx_d.at[j]],
                                 ss[h], add=True)
        return carry

    lax.fori_loop(0, n_chunks, body, 0)
    for sem in ss:
        for _ in range(2):
            pltpu.make_async_copy(rows.at[0], acc.at[idx_d.at[0]],
                                  sem).wait()


def _make_colsplit_kernel():
    @functools.partial(
        pl.kernel,
        out_type=jax.ShapeDtypeStruct((NC, NROWS, DHH), jnp.float32),
        mesh=_sc_mesh(),
        compiler_params=pltpu.CompilerParams(use_tc_tiling_on_sc=False),
        scratch_types=[
            pltpu.VMEM((NCH2, CH), jnp.int32),
            pltpu.VMEM((NCH2, CH), jnp.int32),
            pltpu.VMEM((4, CH, DHH), jnp.float32),
            pltpu.VMEM_SHARED((NROWS, DHH), jnp.float32),
        ] + [pltpu.SemaphoreType.DMA] * 4,
    )
    def gs_kernel(table2, srcp, dstp, zeros, out, idx_s, idx_d, rows, acc,
                  *sems):
        c = lax.axis_index("c")
        s = lax.axis_index("s")
        pltpu.sync_copy(srcp.at[s], idx_s)
        pltpu.sync_copy(dstp.at[s], idx_d)
        pltpu.sync_copy(zeros, acc.at[pl.ds(s * STRIPE, STRIPE)])
        plsc.subcore_barrier()
        _pipelined_chunk_loop(table2.at[c], idx_s, idx_d, rows, acc, sems,
                              NCH2)
        plsc.subcore_barrier()
        pltpu.sync_copy(acc.at[pl.ds(s * STRIPE, STRIPE)],
                        out.at[c, pl.ds(s * STRIPE, STRIPE)])

    return gs_kernel


def _make_edgesplit_kernel(D):
    @functools.partial(
        pl.kernel,
        out_type=jax.ShapeDtypeStruct((NC, NROWS, D), jnp.float32),
        mesh=_sc_mesh(),
        compiler_params=pltpu.CompilerParams(use_tc_tiling_on_sc=False),
        scratch_types=[
            pltpu.VMEM((NCH, CH), jnp.int32),
            pltpu.VMEM((NCH, CH), jnp.int32),
            pltpu.VMEM((4, CH, D), jnp.float32),
            pltpu.VMEM_SHARED((NROWS, D), jnp.float32),
        ] + [pltpu.SemaphoreType.DMA] * 4,
    )
    def gs_kernel(table, srcp, dstp, zeros, out, idx_s, idx_d, rows, acc,
                  *sems):
        c = lax.axis_index("c")
        s = lax.axis_index("s")
        w = c * NS + s
        pltpu.sync_copy(srcp.at[w], idx_s)
        pltpu.sync_copy(dstp.at[w], idx_d)
        pltpu.sync_copy(zeros, acc.at[pl.ds(s * STRIPE, STRIPE)])
        plsc.subcore_barrier()
        _pipelined_chunk_loop(table, idx_s, idx_d, rows, acc, sems, NCH)
        plsc.subcore_barrier()
        pltpu.sync_copy(acc.at[pl.ds(s * STRIPE, STRIPE)],
                        out.at[c, pl.ds(s * STRIPE, STRIPE)])

    return gs_kernel


def _dinv_block(deg_ref):
    deg = deg_ref[0, :, 0:1] + deg_ref[1, :, 0:1]
    return lax.rsqrt(deg)


def _tc1_body(deg_ref, x_ref, w_ref, h_ref, hp_ref):
    dinv = _dinv_block(deg_ref)
    h = jnp.dot(x_ref[...], w_ref[...], preferred_element_type=jnp.float32)
    h_ref[...] = h
    hp = h * dinv
    hp_ref[0] = hp[:, :DHH]
    hp_ref[1] = hp[:, DHH:]


def _tc2_body(deg_ref, s_ref, h1_ref, b_ref, w_ref, h2_ref, hp2_ref):
    dinv = _dinv_block(deg_ref)
    sagg = jnp.concatenate([s_ref[0], s_ref[1]], axis=1)
    a = dinv * sagg + (dinv * dinv) * h1_ref[...] + b_ref[...]
    a = jnp.maximum(a, 0.0)
    h2 = jnp.dot(a, w_ref[...], preferred_element_type=jnp.float32)
    h2_ref[...] = h2
    hp2_ref[...] = h2 * dinv


def _tc3_body(deg_ref, s_ref, h2_ref, b_ref, o_ref):
    dinv = _dinv_block(deg_ref)
    sagg = s_ref[0] + s_ref[1]
    z = dinv * sagg + (dinv * dinv) * h2_ref[...] + b_ref[...]
    m = jnp.max(z, axis=1, keepdims=True)
    ez = jnp.exp(z - m)
    lse = jnp.log(jnp.sum(ez, axis=1, keepdims=True)) + m
    o_ref[...] = z - lse


def _deg_spec():
    return pl.BlockSpec((NC, RB, DEGW), lambda i: (0, i, 0))


def _acc_spec(D):
    return pl.BlockSpec((NC, RB, D), lambda i: (0, i, 0))


def _full_spec(r, c):
    return pl.BlockSpec((r, c), lambda i: (0, 0))


def _row_spec(D):
    return pl.BlockSpec((RB, D), lambda i: (i, 0))


def kernel(x, edge_index, W1, b1, W2, b2):
    src = edge_index[0].astype(jnp.int32)
    dst = edge_index[1].astype(jnp.int32)
    npad = NEP - NE
    pad = jnp.arange(npad, dtype=jnp.int32)
    src_pad = jnp.concatenate([src, pad % N])
    dst_pad = jnp.concatenate([dst, N + pad % (NROWS - N)])
    src_p = src_pad.reshape(NW, NCH, CH)
    dst_p = dst_pad.reshape(NW, NCH, CH)
    src_p2 = src_pad.reshape(NS, NCH2, CH)
    dst_p2 = dst_pad.reshape(NS, NCH2, CH)

    ones = jnp.ones((CH, DEGW), jnp.float32)
    zeros_deg = jnp.zeros((STRIPE, DEGW), jnp.float32)
    zeros_h = jnp.zeros((STRIPE, DHH), jnp.float32)
    zeros_o = jnp.zeros((STRIPE, DOUT), jnp.float32)

    degp = _make_degree_kernel()(dst_p, ones, zeros_deg)

    grid = (N // RB,)

    h1, hp1 = pl.pallas_call(
        _tc1_body,
        grid=grid,
        in_specs=[_deg_spec(), _row_spec(DIN), _full_spec(DIN, DH)],
        out_specs=[_row_spec(DH),
                   pl.BlockSpec((NC, RB, DHH), lambda i: (0, i, 0))],
        out_shape=[jax.ShapeDtypeStruct((N, DH), jnp.float32),
                   jax.ShapeDtypeStruct((NC, N, DHH), jnp.float32)],
    )(degp, x, W1)

    s1p = _make_colsplit_kernel()(hp1, src_p2, dst_p2, zeros_h)

    h2, hp2 = pl.pallas_call(
        _tc2_body,
        grid=grid,
        in_specs=[_deg_spec(), _acc_spec(DHH), _row_spec(DH),
                  _full_spec(1, DH), _full_spec(DH, DOUT)],
        out_specs=[_row_spec(DOUT), _row_spec(DOUT)],
        out_shape=[jax.ShapeDtypeStruct((N, DOUT), jnp.float32)] * 2,
    )(degp, s1p, h1, b1.reshape(1, DH), W2)

    s2p = _make_edgesplit_kernel(DOUT)(hp2, src_p, dst_p, zeros_o)

    out = pl.pallas_call(
        _tc3_body,
        grid=grid,
        in_specs=[_deg_spec(), _acc_spec(DOUT), _row_spec(DOUT),
                  _full_spec(1, DOUT)],
        out_specs=_row_spec(DOUT),
        out_shape=jax.ShapeDtypeStruct((N, DOUT), jnp.float32),
    )(degp, s2p, h2, b2.reshape(1, DOUT))

    return out
